# baseline (speedup 1.0000x reference)
import functools

import jax
import jax.numpy as jnp
from jax import lax
from jax.experimental import pallas as pl
from jax.experimental.pallas import tpu as pltpu
from jax.experimental.pallas import tpu_sc as plsc


def kernel(h, edge_index):
    n_nodes, d = h.shape
    n_edges = edge_index.shape[1]

    info = plsc.get_sparse_core_info()
    nc, ns, L = info.num_cores, info.num_subcores, info.num_lanes
    nw = nc * ns

    assert n_edges % nw == 0
    epw = n_edges // nw
    C = 400
    assert epw % C == 0 and C % L == 0
    n_chunks = epw // C

    src = edge_index[0]
    dst = edge_index[1]

    mesh = plsc.VectorSubcoreMesh(core_axis_name="c", subcore_axis_name="s")

    @functools.partial(
        pl.kernel,
        mesh=mesh,
        out_type=jax.ShapeDtypeStruct((n_edges,), jnp.float32),
        scratch_types=[
            pltpu.VMEM((C,), jnp.int32),
            pltpu.VMEM((C,), jnp.int32),
            pltpu.VMEM((C, d), jnp.float32),
            pltpu.VMEM((C, d), jnp.float32),
            pltpu.VMEM((C,), jnp.float32),
            pltpu.SemaphoreType.DMA,
        ],
        compiler_params=pltpu.CompilerParams(needs_layout_passes=False),
    )
    def ker(h_hbm, src_hbm, dst_hbm, out_hbm, idx_s, idx_d, rows_s, rows_d,
            out_v, sem):
        wid = lax.axis_index("s") * nc + lax.axis_index("c")
        base = wid * epw

        lane = lax.iota(jnp.int32, L)
        last_lane = lane == (L - 1)

        def chunk_body(i, carry):
            off = pl.multiple_of(base + i * C, C)
            pltpu.sync_copy(src_hbm.at[pl.ds(off, C)], idx_s)
            pltpu.sync_copy(dst_hbm.at[pl.ds(off, C)], idx_d)
            cp_s = pltpu.async_copy(h_hbm.at[idx_s], rows_s, sem)
            cp_d = pltpu.async_copy(h_hbm.at[idx_d], rows_d, sem)
            cp_s.wait()
            cp_d.wait()

            @plsc.parallel_loop(0, C)
            def edge_body(e):
                vals = [rows_s[e, pl.ds(k * L, L)] * rows_d[e, pl.ds(k * L, L)]
                        for k in range(d // L)]
                while len(vals) > 1:
                    vals = [vals[i] + vals[i + 1]
                            for i in range(0, len(vals), 2)]
                c = plsc.cumsum(vals[0])
                ev = jnp.full((L,), e, dtype=jnp.int32)
                plsc.store_scatter(out_v, [ev], c, mask=last_lane)

            @plsc.parallel_loop(0, C // L)
            def sig_body(g):
                x = out_v[pl.ds(g * L, L)]
                out_v[pl.ds(g * L, L)] = 1.0 / (1.0 + jnp.exp(-x))
            pltpu.sync_copy(out_v, out_hbm.at[pl.ds(off, C)])
            return carry

        lax.fori_loop(0, n_chunks, chunk_body, 0)

    return ker(h, src, dst)


# speedup vs baseline: 1.1963x; 1.1963x over previous
import functools

import jax
import jax.numpy as jnp
from jax import lax
from jax.experimental import pallas as pl
from jax.experimental.pallas import tpu as pltpu
from jax.experimental.pallas import tpu_sc as plsc


def kernel(h, edge_index):
    n_nodes, d = h.shape
    n_edges = edge_index.shape[1]

    info = plsc.get_sparse_core_info()
    nc, ns, L = info.num_cores, info.num_subcores, info.num_lanes
    nw = nc * ns

    assert n_edges % nw == 0
    epw = n_edges // nw
    C = 80
    assert epw % C == 0 and C % L == 0
    n_chunks = epw // C

    src = edge_index[0]
    dst = edge_index[1]

    mesh = plsc.VectorSubcoreMesh(core_axis_name="c", subcore_axis_name="s")

    @functools.partial(
        pl.kernel,
        mesh=mesh,
        out_type=jax.ShapeDtypeStruct((n_edges,), jnp.float32),
        scratch_types=[
            pltpu.VMEM((epw,), jnp.int32),
            pltpu.VMEM((epw,), jnp.int32),
            pltpu.VMEM((2, C, d), jnp.float32),
            pltpu.VMEM((2, C, d), jnp.float32),
            pltpu.VMEM((epw,), jnp.float32),
            pltpu.SemaphoreType.DMA,
        ],
        compiler_params=pltpu.CompilerParams(needs_layout_passes=False),
    )
    def ker(h_hbm, src_hbm, dst_hbm, out_hbm, ids_s, ids_d, rows_s, rows_d,
            out_v, semr):
        wid = lax.axis_index("s") * nc + lax.axis_index("c")
        base = wid * epw

        lane = lax.iota(jnp.int32, L)
        last_lane = lane == (L - 1)

        pltpu.sync_copy(src_hbm.at[pl.ds(base, epw)], ids_s)
        pltpu.sync_copy(dst_hbm.at[pl.ds(base, epw)], ids_d)

        def start_gather(i, b):
            pltpu.async_copy(h_hbm.at[ids_s.at[pl.ds(i * C, C)]],
                             rows_s.at[b], semr)
            pltpu.async_copy(h_hbm.at[ids_d.at[pl.ds(i * C, C)]],
                             rows_d.at[b], semr)

        def wait_gather(b):
            pltpu.make_async_copy(h_hbm.at[ids_s.at[pl.ds(0, C)]],
                                  rows_s.at[b], semr).wait()
            pltpu.make_async_copy(h_hbm.at[ids_d.at[pl.ds(0, C)]],
                                  rows_d.at[b], semr).wait()

        def compute(i, b):
            base_e = i * C

            @plsc.parallel_loop(0, C)
            def edge_body(e):
                vals = [rows_s[b, e, pl.ds(k * L, L)]
                        * rows_d[b, e, pl.ds(k * L, L)]
                        for k in range(d // L)]
                while len(vals) > 1:
                    vals = [vals[j] + vals[j + 1]
                            for j in range(0, len(vals), 2)]
                c = plsc.cumsum(vals[0])
                ev = jnp.full((L,), base_e + e, dtype=jnp.int32)
                plsc.store_scatter(out_v, [ev], c, mask=last_lane)

            @plsc.parallel_loop(0, C // L)
            def sig_body(g):
                sl = pl.ds(base_e + g * L, L)
                x = out_v[sl]
                out_v[sl] = 1.0 / (1.0 + jnp.exp(-x))

        start_gather(0, 0)

        def chunk_body(i, carry):
            b = lax.rem(i, 2)
            wait_gather(b)
            start_gather(i + 1, 1 - b)
            compute(i, b)
            return carry

        lax.fori_loop(0, n_chunks - 1, chunk_body, 0)

        bl = (n_chunks - 1) % 2
        wait_gather(bl)
        compute(n_chunks - 1, bl)

        pltpu.sync_copy(out_v, out_hbm.at[pl.ds(base, epw)])

    return ker(h, src, dst)


# speedup vs baseline: 1.7619x; 1.4727x over previous
import functools

import jax
import jax.numpy as jnp
from jax import lax
from jax.experimental import pallas as pl
from jax.experimental.pallas import tpu as pltpu
from jax.experimental.pallas import tpu_sc as plsc


def kernel(h, edge_index):
    n_nodes, d = h.shape
    n_edges = edge_index.shape[1]

    info = plsc.get_sparse_core_info()
    nc, ns, L = info.num_cores, info.num_subcores, info.num_lanes
    nw = nc * ns

    assert n_edges % nw == 0
    epw = n_edges // nw
    C = 64
    assert C % L == 0 and epw % 8 == 0 and C % 8 == 0
    n_chunks = (epw + C - 1) // C

    src = edge_index[0]
    dst = edge_index[1]

    mesh = plsc.VectorSubcoreMesh(core_axis_name="c", subcore_axis_name="s")

    @functools.partial(
        pl.kernel,
        mesh=mesh,
        out_type=jax.ShapeDtypeStruct((n_edges,), jnp.float32),
        scratch_types=[
            pltpu.VMEM((2, C), jnp.int32),
            pltpu.VMEM((2, C), jnp.int32),
            pltpu.VMEM((2, C, d), jnp.float32),
            pltpu.VMEM((2, C, d), jnp.float32),
            pltpu.VMEM((epw,), jnp.float32),
            pltpu.VMEM_SHARED((n_nodes, d), jnp.float32),
            pltpu.SemaphoreType.DMA,
            pltpu.SemaphoreType.DMA,
        ],
        compiler_params=pltpu.CompilerParams(needs_layout_passes=False),
    )
    def ker(h_hbm, src_hbm, dst_hbm, out_hbm, idw_s, idw_d, rows_s, rows_d,
            out_v, h_sh, semr, semi):
        wid = lax.axis_index("s") * nc + lax.axis_index("c")
        sid = lax.axis_index("s")
        base = wid * epw

        lane = lax.iota(jnp.int32, L)
        last_lane = lane == (L - 1)

        rpt = (n_nodes // ns) & ~7
        tail = n_nodes - (ns - 1) * rpt

        @pl.when(sid < ns - 1)
        def _():
            pltpu.sync_copy(h_hbm.at[pl.ds(sid * rpt, rpt)],
                            h_sh.at[pl.ds(sid * rpt, rpt)])

        @pl.when(sid == ns - 1)
        def _():
            pltpu.sync_copy(h_hbm.at[pl.ds((ns - 1) * rpt, tail)],
                            h_sh.at[pl.ds((ns - 1) * rpt, tail)])

        plsc.subcore_barrier()

        def off(i):
            return pl.multiple_of(base + jnp.minimum(i * C, epw - C), 8)

        def start_ids(i, b):
            pltpu.async_copy(src_hbm.at[pl.ds(off(i), C)], idw_s.at[b], semi)
            pltpu.async_copy(dst_hbm.at[pl.ds(off(i), C)], idw_d.at[b], semi)

        def wait_ids(b):
            pltpu.make_async_copy(src_hbm.at[pl.ds(base, C)],
                                  idw_s.at[b], semi).wait()
            pltpu.make_async_copy(dst_hbm.at[pl.ds(base, C)],
                                  idw_d.at[b], semi).wait()

        def start_gather(b):
            pltpu.async_copy(h_sh.at[idw_s.at[b]], rows_s.at[b], semr)
            pltpu.async_copy(h_sh.at[idw_d.at[b]], rows_d.at[b], semr)

        def wait_gather(b):
            pltpu.make_async_copy(h_sh.at[idw_s.at[b]],
                                  rows_s.at[b], semr).wait()
            pltpu.make_async_copy(h_sh.at[idw_d.at[b]],
                                  rows_d.at[b], semr).wait()

        def compute(i, b):
            chunk0 = jnp.minimum(i * C, epw - C)

            @plsc.parallel_loop(0, C)
            def edge_body(e):
                vals = [rows_s[b, e, pl.ds(k * L, L)]
                        * rows_d[b, e, pl.ds(k * L, L)]
                        for k in range(d // L)]
                while len(vals) > 1:
                    vals = [vals[j] + vals[j + 1]
                            for j in range(0, len(vals), 2)]
                c = plsc.cumsum(vals[0])
                ev = jnp.full((L,), chunk0 + e, dtype=jnp.int32)
                plsc.store_scatter(out_v, [ev], c, mask=last_lane)

            @plsc.parallel_loop(0, C // L)
            def sig_body(g):
                sl = pl.ds(chunk0 + g * L, L)
                x = out_v[sl]
                out_v[sl] = 1.0 / (1.0 + jnp.exp(-x))

        start_ids(0, 0)
        start_ids(1, 1)
        wait_ids(0)
        start_gather(0)

        def chunk_body(i, carry):
            b = lax.rem(i, 2)
            wait_gather(b)
            wait_ids(1 - b)
            start_gather(1 - b)
            start_ids(i + 2, b)
            compute(i, b)
            return carry

        lax.fori_loop(0, n_chunks, chunk_body, 0)

        wait_gather(n_chunks % 2)
        wait_ids(n_chunks % 2)

        pltpu.sync_copy(out_v, out_hbm.at[pl.ds(base, epw)])

    return ker(h, src, dst)


# speedup vs baseline: 1.8179x; 1.0318x over previous
import functools

import jax
import jax.numpy as jnp
from jax import lax
from jax.experimental import pallas as pl
from jax.experimental.pallas import tpu as pltpu
from jax.experimental.pallas import tpu_sc as plsc


def kernel(h, edge_index):
    n_nodes, d = h.shape
    n_edges = edge_index.shape[1]

    info = plsc.get_sparse_core_info()
    nc, ns, L = info.num_cores, info.num_subcores, info.num_lanes
    nw = nc * ns

    assert n_edges % nw == 0
    epw = n_edges // nw
    C = 64
    assert C % L == 0 and epw % 8 == 0 and C % 8 == 0
    n_chunks = (epw + C - 1) // C

    src = edge_index[0]
    dst = edge_index[1]

    mesh = plsc.VectorSubcoreMesh(core_axis_name="c", subcore_axis_name="s")

    @functools.partial(
        pl.kernel,
        mesh=mesh,
        out_type=jax.ShapeDtypeStruct((n_edges,), jnp.float32),
        scratch_types=[
            pltpu.VMEM((2, C), jnp.int32),
            pltpu.VMEM((2, C), jnp.int32),
            pltpu.VMEM((2, C, d), jnp.float32),
            pltpu.VMEM((2, C, d), jnp.float32),
            pltpu.VMEM((epw,), jnp.float32),
            pltpu.VMEM_SHARED((n_nodes, d), jnp.float32),
            pltpu.SemaphoreType.DMA,
            pltpu.SemaphoreType.DMA,
        ],
        compiler_params=pltpu.CompilerParams(needs_layout_passes=False),
    )
    def ker(h_hbm, src_hbm, dst_hbm, out_hbm, idw_s, idw_d, rows_s, rows_d,
            out_v, h_sh, semr, semi):
        wid = lax.axis_index("s") * nc + lax.axis_index("c")
        sid = lax.axis_index("s")
        base = wid * epw

        lane = lax.iota(jnp.int32, L)
        last_lane = lane == (L - 1)

        rpt = (n_nodes // ns) & ~7
        tail = n_nodes - (ns - 1) * rpt

        @pl.when(sid < ns - 1)
        def _():
            pltpu.sync_copy(h_hbm.at[pl.ds(sid * rpt, rpt)],
                            h_sh.at[pl.ds(sid * rpt, rpt)])

        @pl.when(sid == ns - 1)
        def _():
            pltpu.sync_copy(h_hbm.at[pl.ds((ns - 1) * rpt, tail)],
                            h_sh.at[pl.ds((ns - 1) * rpt, tail)])

        plsc.subcore_barrier()

        def off(i):
            return pl.multiple_of(base + jnp.minimum(i * C, epw - C), 8)

        def start_ids(i, b):
            pltpu.async_copy(src_hbm.at[pl.ds(off(i), C)], idw_s.at[b], semi)
            pltpu.async_copy(dst_hbm.at[pl.ds(off(i), C)], idw_d.at[b], semi)

        def wait_ids(b):
            pltpu.make_async_copy(src_hbm.at[pl.ds(base, C)],
                                  idw_s.at[b], semi).wait()
            pltpu.make_async_copy(dst_hbm.at[pl.ds(base, C)],
                                  idw_d.at[b], semi).wait()

        def start_gather(b):
            pltpu.async_copy(h_sh.at[idw_s.at[b]], rows_s.at[b], semr)
            pltpu.async_copy(h_sh.at[idw_d.at[b]], rows_d.at[b], semr)

        def wait_gather(b):
            pltpu.make_async_copy(h_sh.at[idw_s.at[b]],
                                  rows_s.at[b], semr).wait()
            pltpu.make_async_copy(h_sh.at[idw_d.at[b]],
                                  rows_d.at[b], semr).wait()

        def compute(i, b):
            chunk0 = jnp.minimum(i * C, epw - C)

            @plsc.parallel_loop(0, C)
            def edge_body(e):
                vals = [rows_s[b, e, pl.ds(k * L, L)]
                        * rows_d[b, e, pl.ds(k * L, L)]
                        for k in range(d // L)]
                while len(vals) > 1:
                    vals = [vals[j] + vals[j + 1]
                            for j in range(0, len(vals), 2)]
                c = plsc.cumsum(vals[0])
                ev = jnp.full((L,), chunk0 + e, dtype=jnp.int32)
                plsc.store_scatter(out_v, [ev], c, mask=last_lane)

            @plsc.parallel_loop(0, C // L)
            def sig_body(g):
                sl = pl.ds(chunk0 + g * L, L)
                x = out_v[sl]
                out_v[sl] = 1.0 / (1.0 + jnp.exp(-x))

        start_ids(0, 0)
        start_ids(1, 1)
        wait_ids(0)
        start_gather(0)

        def chunk_body(i, carry):
            b = lax.rem(i, 2)
            wait_ids(1 - b)
            start_gather(1 - b)
            wait_gather(b)
            start_ids(i + 2, b)
            compute(i, b)
            return carry

        lax.fori_loop(0, n_chunks, chunk_body, 0)

        wait_gather(n_chunks % 2)
        wait_ids(n_chunks % 2)

        pltpu.sync_copy(out_v, out_hbm.at[pl.ds(base, epw)])

    return ker(h, src, dst)


# speedup vs baseline: 1.8313x; 1.0074x over previous
import functools

import jax
import jax.numpy as jnp
from jax import lax
from jax.experimental import pallas as pl
from jax.experimental.pallas import tpu as pltpu
from jax.experimental.pallas import tpu_sc as plsc


def kernel(h, edge_index):
    n_nodes, d = h.shape
    n_edges = edge_index.shape[1]

    info = plsc.get_sparse_core_info()
    nc, ns, L = info.num_cores, info.num_subcores, info.num_lanes
    nw = nc * ns

    assert n_edges % nw == 0
    epw = n_edges // nw
    C = 48
    NB = 3
    assert C % L == 0 and epw % 8 == 0 and C % 8 == 0
    n_chunks = (epw + C - 1) // C
    assert n_chunks >= NB

    src = edge_index[0]
    dst = edge_index[1]

    mesh = plsc.VectorSubcoreMesh(core_axis_name="c", subcore_axis_name="s")

    @functools.partial(
        pl.kernel,
        mesh=mesh,
        out_type=jax.ShapeDtypeStruct((n_edges,), jnp.float32),
        scratch_types=[
            pltpu.VMEM((NB, C), jnp.int32),
            pltpu.VMEM((NB, C), jnp.int32),
            pltpu.VMEM((NB, C, d), jnp.float32),
            pltpu.VMEM((NB, C, d), jnp.float32),
            pltpu.VMEM((epw,), jnp.float32),
            pltpu.VMEM_SHARED((n_nodes, d), jnp.float32),
            pltpu.SemaphoreType.DMA,
            pltpu.SemaphoreType.DMA,
        ],
        compiler_params=pltpu.CompilerParams(needs_layout_passes=False),
    )
    def ker(h_hbm, src_hbm, dst_hbm, out_hbm, idw_s, idw_d, rows_s, rows_d,
            out_v, h_sh, semr, semi):
        wid = lax.axis_index("s") * nc + lax.axis_index("c")
        sid = lax.axis_index("s")
        base = wid * epw

        lane = lax.iota(jnp.int32, L)
        last_lane = lane == (L - 1)

        rpt = (n_nodes // ns) & ~7
        tail = n_nodes - (ns - 1) * rpt

        @pl.when(sid < ns - 1)
        def _():
            pltpu.sync_copy(h_hbm.at[pl.ds(sid * rpt, rpt)],
                            h_sh.at[pl.ds(sid * rpt, rpt)])

        @pl.when(sid == ns - 1)
        def _():
            pltpu.sync_copy(h_hbm.at[pl.ds((ns - 1) * rpt, tail)],
                            h_sh.at[pl.ds((ns - 1) * rpt, tail)])

        plsc.subcore_barrier()

        def off(i):
            return pl.multiple_of(base + jnp.minimum(i * C, epw - C), 8)

        def start_ids(i, b):
            pltpu.async_copy(src_hbm.at[pl.ds(off(i), C)], idw_s.at[b], semi)
            pltpu.async_copy(dst_hbm.at[pl.ds(off(i), C)], idw_d.at[b], semi)

        def wait_ids(b):
            pltpu.make_async_copy(src_hbm.at[pl.ds(base, C)],
                                  idw_s.at[b], semi).wait()
            pltpu.make_async_copy(dst_hbm.at[pl.ds(base, C)],
                                  idw_d.at[b], semi).wait()

        def start_gather(b):
            pltpu.async_copy(h_sh.at[idw_s.at[b]], rows_s.at[b], semr)
            pltpu.async_copy(h_sh.at[idw_d.at[b]], rows_d.at[b], semr)

        def wait_gather(b):
            pltpu.make_async_copy(h_sh.at[idw_s.at[b]],
                                  rows_s.at[b], semr).wait()
            pltpu.make_async_copy(h_sh.at[idw_d.at[b]],
                                  rows_d.at[b], semr).wait()

        def compute(i, b):
            chunk0 = jnp.minimum(i * C, epw - C)

            @plsc.parallel_loop(0, C)
            def edge_body(e):
                vals = [rows_s[b, e, pl.ds(k * L, L)]
                        * rows_d[b, e, pl.ds(k * L, L)]
                        for k in range(d // L)]
                while len(vals) > 1:
                    vals = [vals[j] + vals[j + 1]
                            for j in range(0, len(vals), 2)]
                c = plsc.cumsum(vals[0])
                ev = jnp.full((L,), chunk0 + e, dtype=jnp.int32)
                plsc.store_scatter(out_v, [ev], c, mask=last_lane)

            @plsc.parallel_loop(0, C // L)
            def sig_body(g):
                sl = pl.ds(chunk0 + g * L, L)
                x = out_v[sl]
                out_v[sl] = 1.0 / (1.0 + jnp.exp(-x))

        start_ids(0, 0)
        start_ids(1, 1)
        start_ids(2, 2)
        wait_ids(0)
        start_gather(0)
        wait_ids(1)
        start_gather(1)

        def chunk_body(i, carry):
            b = lax.rem(i, NB)
            b2 = lax.rem(i + 2, NB)
            wait_ids(b2)
            start_gather(b2)
            wait_gather(b)
            start_ids(i + NB, b)
            compute(i, b)
            return carry

        lax.fori_loop(0, n_chunks, chunk_body, 0)

        wait_gather(n_chunks % NB)
        wait_gather((n_chunks + 1) % NB)
        wait_ids(n_chunks % NB)

        pltpu.sync_copy(out_v, out_hbm.at[pl.ds(base, epw)])

    return ker(h, src, dst)


# speedup vs baseline: 1.8373x; 1.0033x over previous
import functools

import jax
import jax.numpy as jnp
from jax import lax
from jax.experimental import pallas as pl
from jax.experimental.pallas import tpu as pltpu
from jax.experimental.pallas import tpu_sc as plsc


def kernel(h, edge_index):
    n_nodes, d = h.shape
    n_edges = edge_index.shape[1]

    info = plsc.get_sparse_core_info()
    nc, ns, L = info.num_cores, info.num_subcores, info.num_lanes
    nw = nc * ns

    assert n_edges % nw == 0
    epw = n_edges // nw
    C = 48
    NB = 3
    assert C % L == 0 and epw % 8 == 0 and C % 8 == 0
    n_chunks = (epw + C - 1) // C
    assert n_chunks >= NB

    src = edge_index[0]
    dst = edge_index[1]

    mesh = plsc.VectorSubcoreMesh(core_axis_name="c", subcore_axis_name="s")

    @functools.partial(
        pl.kernel,
        mesh=mesh,
        out_type=jax.ShapeDtypeStruct((n_edges,), jnp.float32),
        scratch_types=[
            pltpu.VMEM((NB, C), jnp.int32),
            pltpu.VMEM((NB, C), jnp.int32),
            pltpu.VMEM((NB, C, d), jnp.float32),
            pltpu.VMEM((NB, C, d), jnp.float32),
            pltpu.VMEM((epw,), jnp.float32),
            pltpu.VMEM_SHARED((n_nodes, d), jnp.float32),
            pltpu.SemaphoreType.DMA,
            pltpu.SemaphoreType.DMA,
        ],
        compiler_params=pltpu.CompilerParams(needs_layout_passes=False),
    )
    def ker(h_hbm, src_hbm, dst_hbm, out_hbm, idw_s, idw_d, rows_s, rows_d,
            out_v, h_sh, semr, semi):
        wid = lax.axis_index("s") * nc + lax.axis_index("c")
        sid = lax.axis_index("s")
        base = wid * epw

        lane = lax.iota(jnp.int32, L)
        last_lane = lane == (L - 1)

        rpt = (n_nodes // ns) & ~7
        tail = n_nodes - (ns - 1) * rpt

        @pl.when(sid < ns - 1)
        def _():
            pltpu.sync_copy(h_hbm.at[pl.ds(sid * rpt, rpt)],
                            h_sh.at[pl.ds(sid * rpt, rpt)])

        @pl.when(sid == ns - 1)
        def _():
            pltpu.sync_copy(h_hbm.at[pl.ds((ns - 1) * rpt, tail)],
                            h_sh.at[pl.ds((ns - 1) * rpt, tail)])

        plsc.subcore_barrier()

        def off(i):
            return pl.multiple_of(base + jnp.minimum(i * C, epw - C), 8)

        def start_ids(i, b):
            pltpu.async_copy(src_hbm.at[pl.ds(off(i), C)], idw_s.at[b], semi)
            pltpu.async_copy(dst_hbm.at[pl.ds(off(i), C)], idw_d.at[b], semi)

        def wait_ids(b):
            pltpu.make_async_copy(src_hbm.at[pl.ds(base, C)],
                                  idw_s.at[b], semi).wait()
            pltpu.make_async_copy(dst_hbm.at[pl.ds(base, C)],
                                  idw_d.at[b], semi).wait()

        def start_gather(b):
            pltpu.async_copy(h_sh.at[idw_s.at[b]], rows_s.at[b], semr)
            pltpu.async_copy(h_sh.at[idw_d.at[b]], rows_d.at[b], semr)

        def wait_gather(b):
            pltpu.make_async_copy(h_sh.at[idw_s.at[b]],
                                  rows_s.at[b], semr).wait()
            pltpu.make_async_copy(h_sh.at[idw_d.at[b]],
                                  rows_d.at[b], semr).wait()

        def compute(i, b):
            chunk0 = jnp.minimum(i * C, epw - C)

            @plsc.parallel_loop(0, C, unroll=2)
            def edge_body(e):
                vals = [rows_s[b, e, pl.ds(k * L, L)]
                        * rows_d[b, e, pl.ds(k * L, L)]
                        for k in range(d // L)]
                while len(vals) > 1:
                    vals = [vals[j] + vals[j + 1]
                            for j in range(0, len(vals), 2)]
                c = plsc.cumsum(vals[0])
                ev = jnp.full((L,), chunk0 + e, dtype=jnp.int32)
                plsc.store_scatter(out_v, [ev], c, mask=last_lane)

            @plsc.parallel_loop(0, C // L)
            def sig_body(g):
                sl = pl.ds(chunk0 + g * L, L)
                x = out_v[sl]
                out_v[sl] = 1.0 / (1.0 + jnp.exp(-x))

        start_ids(0, 0)
        start_ids(1, 1)
        start_ids(2, 2)
        wait_ids(0)
        start_gather(0)
        wait_ids(1)
        start_gather(1)

        def chunk_body(i, carry):
            b = lax.rem(i, NB)
            b2 = lax.rem(i + 2, NB)
            wait_ids(b2)
            start_gather(b2)
            wait_gather(b)
            start_ids(i + NB, b)
            compute(i, b)
            return carry

        lax.fori_loop(0, n_chunks, chunk_body, 0)

        wait_gather(n_chunks % NB)
        wait_gather((n_chunks + 1) % NB)
        wait_ids(n_chunks % NB)

        pltpu.sync_copy(out_v, out_hbm.at[pl.ds(base, epw)])

    return ker(h, src, dst)
